# baseline (speedup 1.0000x reference)
import functools

import jax
import jax.numpy as jnp
from jax import lax
from jax.experimental import pallas as pl
from jax.experimental.pallas import tpu as pltpu
from jax.experimental.pallas import tpu_sc as plsc

_NC = 2
_NS = 16


def _make_sc_scatter_add(N, D, E, C, pipelined=True):
    NW = _NC * _NS
    epw = E // NW
    nch = epw // C
    assert epw * NW == E and nch * C == epw and C <= 128
    NT = 10
    RPT = N // NT
    assert RPT * NT == N and RPT % 8 == 0

    mesh = plsc.VectorSubcoreMesh(core_axis_name="c", subcore_axis_name="s")

    @functools.partial(
        pl.kernel,
        out_type=jax.ShapeDtypeStruct((_NC, N, D), jnp.float32),
        mesh=mesh,
        compiler_params=pltpu.CompilerParams(use_tc_tiling_on_sc=False),
        scratch_types=[
            pltpu.VMEM((nch, C), jnp.int32),
            pltpu.VMEM((nch, C), jnp.int32),
            pltpu.VMEM((2, C, D), jnp.float32),
            pltpu.VMEM_SHARED((N, D), jnp.float32),
        ] + ([pltpu.SemaphoreType.DMA] if pipelined else []),
    )
    def k(table_hbm, src_hbm, dst_hbm, zeros_hbm, out_hbm,
          src_v, dst_v, buf, aggr, *maybe_sem):
        c = lax.axis_index("c")
        s = lax.axis_index("s")
        wid = c * _NS + s
        row0 = pl.multiple_of(s * RPT, 8)

        sl = pl.ds(row0, RPT)

        @pl.when((s < NT) & (c == 0))
        def _():
            pltpu.sync_copy(table_hbm.at[sl], aggr.at[sl])

        @pl.when((s < NT) & (c != 0))
        def _():
            pltpu.sync_copy(zeros_hbm.at[sl], aggr.at[sl])

        pltpu.sync_copy(src_hbm.at[wid], src_v)
        pltpu.sync_copy(dst_hbm.at[wid], dst_v)
        plsc.subcore_barrier()

        if pipelined:
            gsem = maybe_sem[0]

            def gather(i):
                pltpu.async_copy(table_hbm.at[src_v.at[i]], buf.at[i % 2],
                                 gsem)

            gather(0)

            def body(i, carry):
                pltpu.make_async_copy(table_hbm.at[src_v.at[0]], buf.at[0],
                                      gsem).wait()

                @pl.when(i + 1 < nch)
                def _():
                    gather(i + 1)

                pltpu.sync_copy(buf.at[i % 2], aggr.at[dst_v.at[i]],
                                add=True)
                return carry
        else:
            def body(i, carry):
                pltpu.sync_copy(table_hbm.at[src_v.at[i]], buf.at[0])
                pltpu.sync_copy(buf.at[0], aggr.at[dst_v.at[i]], add=True)
                return carry

        lax.fori_loop(0, nch, body, 0)

        plsc.subcore_barrier()

        @pl.when(s < NT)
        def _():
            pltpu.sync_copy(aggr.at[sl], out_hbm.at[c].at[sl])

    return k


def _mlp1_body(plo0, plo1, phi0, phi1, Wlo, Whi, ba, Wb, bb, out):
    hlo = plo0[...] + plo1[...]
    hhi = phi0[...] + phi1[...]
    h = (jnp.dot(hlo, Wlo[...], preferred_element_type=jnp.float32)
         + jnp.dot(hhi, Whi[...], preferred_element_type=jnp.float32)
         + ba[...])
    h = jnp.maximum(h, 0.0)
    h = jnp.dot(h, Wb[...], preferred_element_type=jnp.float32) + bb[...]
    out[...] = jnp.maximum(h, 0.0)


def _mlp2_body(p0, p1, Wa, ba, Wb, bb, Wl1, bl1, Wl2, bl2, out):
    h = p0[...] + p1[...]
    h = jnp.maximum(jnp.dot(h, Wa[...], preferred_element_type=jnp.float32)
                    + ba[...], 0.0)
    h = jnp.dot(h, Wb[...], preferred_element_type=jnp.float32) + bb[...]
    h = jnp.maximum(h, 0.0)
    h = jnp.maximum(jnp.dot(h, Wl1[...], preferred_element_type=jnp.float32)
                    + bl1[...], 0.0)
    h = jnp.maximum(jnp.dot(h, Wl2[...], preferred_element_type=jnp.float32)
                    + bl2[...], 0.0)
    out[...] = jax.nn.sigmoid(h)


def _full(shape):
    return pl.BlockSpec(shape, lambda i: (0, 0))


def _mlp1(plo, phi, Wa, ba, Wb, bb, B=1000):
    N, H = plo.shape[1], plo.shape[2]
    return pl.pallas_call(
        _mlp1_body,
        grid=(N // B,),
        in_specs=[pl.BlockSpec((B, H), lambda i: (i, 0))] * 4 +
                 [_full((H, H)), _full((H, H)), _full((1, H)),
                  _full((H, H)), _full((1, H))],
        out_specs=pl.BlockSpec((B, H), lambda i: (i, 0)),
        out_shape=jax.ShapeDtypeStruct((N, H), jnp.float32),
    )(plo[0], plo[1], phi[0], phi[1], Wa[:H], Wa[H:],
      ba.reshape(1, -1), Wb, bb.reshape(1, -1))


def _mlp2(p, Wa, ba, Wb, bb, Wl1, bl1, Wl2, bl2, B=1000):
    N, H = p.shape[1], p.shape[2]
    K = Wl1.shape[1]
    return pl.pallas_call(
        _mlp2_body,
        grid=(N // B,),
        in_specs=[pl.BlockSpec((B, H), lambda i: (i, 0)),
                  pl.BlockSpec((B, H), lambda i: (i, 0)),
                  _full((H, H)), _full((1, H)),
                  _full((H, H)), _full((1, H)),
                  _full((H, K)), _full((1, K)),
                  _full((K, 1)), _full((1, 1))],
        out_specs=pl.BlockSpec((B, 1), lambda i: (i, 0)),
        out_shape=jax.ShapeDtypeStruct((N, 1), jnp.float32),
    )(p[0], p[1], Wa, ba.reshape(1, -1), Wb, bb.reshape(1, -1),
      Wl1, bl1.reshape(1, -1), Wl2, bl2.reshape(1, -1))


def kernel(x, edge_index, batch, W1a, b1a, W1b, b1b, W2a, b2a, W2b, b2b,
           Wl1, bl1, Wl2, bl2):
    N, D = x.shape
    E = edge_index.shape[1]
    H = W1a.shape[1]
    C = 100
    NW = _NC * _NS
    src3d = edge_index[0].reshape(NW, E // (NW * C), C)
    dst3d = edge_index[1].reshape(NW, E // (NW * C), C)

    zeros_h = jnp.zeros((N, H), jnp.float32)

    scat = _make_sc_scatter_add(N, H, E, C)

    x_lo = lax.slice(x, (0, 0), (N, H))
    x_hi = lax.slice(x, (0, H), (N, D))
    p_lo = scat(x_lo, src3d, dst3d, zeros_h)
    p_hi = scat(x_hi, src3d, dst3d, zeros_h)
    h1 = _mlp1(p_lo, p_hi, W1a, b1a, W1b, b1b)
    p2 = scat(h1, src3d, dst3d, zeros_h)
    return _mlp2(p2, W2a, b2a, W2b, b2b, Wl1, bl1, Wl2, bl2)


# speedup vs baseline: 1.1348x; 1.1348x over previous
import functools

import jax
import jax.numpy as jnp
from jax import lax
from jax.experimental import pallas as pl
from jax.experimental.pallas import tpu as pltpu
from jax.experimental.pallas import tpu_sc as plsc

_NC = 2
_NS = 16


def _make_sc_scatter_add(N, D, E, C, tc_tiling):
    NW = _NC * _NS
    epw = E // NW
    nch = epw // C
    assert epw * NW == E and nch * C == epw and C <= 128
    NT = 10
    RPT = N // NT
    assert RPT * NT == N and RPT % 8 == 0

    mesh = plsc.VectorSubcoreMesh(core_axis_name="c", subcore_axis_name="s")

    @functools.partial(
        pl.kernel,
        out_type=jax.ShapeDtypeStruct((_NC, N, D), jnp.float32),
        mesh=mesh,
        compiler_params=pltpu.CompilerParams(use_tc_tiling_on_sc=tc_tiling),
        scratch_types=[
            pltpu.VMEM((nch, C), jnp.int32),
            pltpu.VMEM((nch, C), jnp.int32),
            pltpu.VMEM((C, D), jnp.float32),
            pltpu.VMEM_SHARED((N, D), jnp.float32),
        ],
    )
    def k(table_hbm, src_hbm, dst_hbm, zeros_hbm, out_hbm,
          src_v, dst_v, buf, aggr):
        c = lax.axis_index("c")
        s = lax.axis_index("s")
        wid = c * _NS + s
        row0 = pl.multiple_of(s * RPT, 8)

        sl = pl.ds(row0, RPT)

        @pl.when((s < NT) & (c == 0))
        def _():
            pltpu.sync_copy(table_hbm.at[sl], aggr.at[sl])

        @pl.when((s < NT) & (c != 0))
        def _():
            pltpu.sync_copy(zeros_hbm.at[sl], aggr.at[sl])

        pltpu.sync_copy(src_hbm.at[wid], src_v)
        pltpu.sync_copy(dst_hbm.at[wid], dst_v)
        plsc.subcore_barrier()

        def body(i, carry):
            pltpu.sync_copy(table_hbm.at[src_v.at[i]], buf)
            pltpu.sync_copy(buf, aggr.at[dst_v.at[i]], add=True)
            return carry

        lax.fori_loop(0, nch, body, 0)

        plsc.subcore_barrier()

        @pl.when(s < NT)
        def _():
            pltpu.sync_copy(aggr.at[sl], out_hbm.at[c].at[sl])

    return k


def _mlp1_body(p0, p1, Wa, ba, Wb, bb, out):
    h = p0[...] + p1[...]
    h = jnp.maximum(jnp.dot(h, Wa[...], preferred_element_type=jnp.float32)
                    + ba[...], 0.0)
    h = jnp.dot(h, Wb[...], preferred_element_type=jnp.float32) + bb[...]
    out[...] = jnp.maximum(h, 0.0)


def _mlp2_body(p0, p1, Wa, ba, Wb, bb, Wl1, bl1, Wl2, bl2, out):
    h = p0[...] + p1[...]
    h = jnp.maximum(jnp.dot(h, Wa[...], preferred_element_type=jnp.float32)
                    + ba[...], 0.0)
    h = jnp.dot(h, Wb[...], preferred_element_type=jnp.float32) + bb[...]
    h = jnp.maximum(h, 0.0)
    h = jnp.maximum(jnp.dot(h, Wl1[...], preferred_element_type=jnp.float32)
                    + bl1[...], 0.0)
    h = jnp.maximum(jnp.dot(h, Wl2[...], preferred_element_type=jnp.float32)
                    + bl2[...], 0.0)
    out[...] = jax.nn.sigmoid(h)


def _full(shape):
    return pl.BlockSpec(shape, lambda i: (0, 0))


def _mlp1(p, Wa, ba, Wb, bb, B=1000):
    N, D = p.shape[1], p.shape[2]
    H = Wa.shape[1]
    return pl.pallas_call(
        _mlp1_body,
        grid=(N // B,),
        in_specs=[pl.BlockSpec((B, D), lambda i: (i, 0)),
                  pl.BlockSpec((B, D), lambda i: (i, 0)),
                  _full((D, H)), _full((1, H)),
                  _full((H, H)), _full((1, H))],
        out_specs=pl.BlockSpec((B, H), lambda i: (i, 0)),
        out_shape=jax.ShapeDtypeStruct((N, H), jnp.float32),
    )(p[0], p[1], Wa, ba.reshape(1, -1), Wb, bb.reshape(1, -1))


def _mlp2(p, Wa, ba, Wb, bb, Wl1, bl1, Wl2, bl2, B=1000):
    N, H = p.shape[1], p.shape[2]
    K = Wl1.shape[1]
    return pl.pallas_call(
        _mlp2_body,
        grid=(N // B,),
        in_specs=[pl.BlockSpec((B, H), lambda i: (i, 0)),
                  pl.BlockSpec((B, H), lambda i: (i, 0)),
                  _full((H, H)), _full((1, H)),
                  _full((H, H)), _full((1, H)),
                  _full((H, K)), _full((1, K)),
                  _full((K, 1)), _full((1, 1))],
        out_specs=pl.BlockSpec((B, 1), lambda i: (i, 0)),
        out_shape=jax.ShapeDtypeStruct((N, 1), jnp.float32),
    )(p[0], p[1], Wa, ba.reshape(1, -1), Wb, bb.reshape(1, -1),
      Wl1, bl1.reshape(1, -1), Wl2, bl2.reshape(1, -1))


def kernel(x, edge_index, batch, W1a, b1a, W1b, b1b, W2a, b2a, W2b, b2b,
           Wl1, bl1, Wl2, bl2):
    N, D = x.shape
    E = edge_index.shape[1]
    H = W1a.shape[1]
    C = 125
    NW = _NC * _NS
    src3d = edge_index[0].reshape(NW, E // (NW * C), C)
    dst3d = edge_index[1].reshape(NW, E // (NW * C), C)

    zeros_d = jnp.zeros((N, D), jnp.float32)
    zeros_h = jnp.zeros((N, H), jnp.float32)

    scat1 = _make_sc_scatter_add(N, D, E, C, tc_tiling=True)
    scat2 = _make_sc_scatter_add(N, H, E, C, tc_tiling=False)

    p1 = scat1(x, src3d, dst3d, zeros_d)
    h1 = _mlp1(p1, W1a, b1a, W1b, b1b)
    p2 = scat2(h1, src3d, dst3d, zeros_h)
    return _mlp2(p2, W2a, b2a, W2b, b2b, Wl1, bl1, Wl2, bl2)


# speedup vs baseline: 1.1602x; 1.0223x over previous
import functools

import jax
import jax.numpy as jnp
from jax import lax
from jax.experimental import pallas as pl
from jax.experimental.pallas import tpu as pltpu
from jax.experimental.pallas import tpu_sc as plsc

_NC = 2
_NS = 16


def _make_sc_scatter_add(N, D, E, C, tc_tiling):
    NW = _NC * _NS
    epw = E // NW
    nch = epw // C
    assert epw * NW == E and nch * C == epw and C <= 128
    NT = 10
    RPT = N // NT
    assert RPT * NT == N and RPT % 8 == 0

    mesh = plsc.VectorSubcoreMesh(core_axis_name="c", subcore_axis_name="s")

    @functools.partial(
        pl.kernel,
        out_type=jax.ShapeDtypeStruct((_NC, N, D), jnp.float32),
        mesh=mesh,
        compiler_params=pltpu.CompilerParams(use_tc_tiling_on_sc=tc_tiling),
        scratch_types=[
            pltpu.VMEM((nch, C), jnp.int32),
            pltpu.VMEM((nch, C), jnp.int32),
            pltpu.VMEM((C, D), jnp.float32),
            pltpu.VMEM_SHARED((N, D), jnp.float32),
        ],
    )
    def k(table_hbm, src_hbm, dst_hbm, out_hbm,
          src_v, dst_v, buf, aggr):
        c = lax.axis_index("c")
        s = lax.axis_index("s")
        wid = c * _NS + s
        row0 = pl.multiple_of(s * RPT, 8)

        sl = pl.ds(row0, RPT)

        @pl.when(s < NT)
        def _():
            pltpu.sync_copy(table_hbm.at[sl], aggr.at[sl])

        pltpu.sync_copy(src_hbm.at[wid], src_v)
        pltpu.sync_copy(dst_hbm.at[wid], dst_v)
        plsc.subcore_barrier()

        def body(i, carry):
            pltpu.sync_copy(table_hbm.at[src_v.at[i]], buf)
            pltpu.sync_copy(buf, aggr.at[dst_v.at[i]], add=True)
            return carry

        lax.fori_loop(0, nch, body, 0)

        plsc.subcore_barrier()

        @pl.when(s < NT)
        def _():
            pltpu.sync_copy(aggr.at[sl], out_hbm.at[c].at[sl])

    return k


def _mlp1_body(p0, p1, xb, Wa, ba, Wb, bb, out):
    h = p0[...] + p1[...] - xb[...]
    h = jnp.maximum(jnp.dot(h, Wa[...], preferred_element_type=jnp.float32)
                    + ba[...], 0.0)
    h = jnp.dot(h, Wb[...], preferred_element_type=jnp.float32) + bb[...]
    out[...] = jnp.maximum(h, 0.0)


def _mlp2_body(p0, p1, xb, Wa, ba, Wb, bb, Wl1, bl1, Wl2, bl2, out):
    h = p0[...] + p1[...] - xb[...]
    h = jnp.maximum(jnp.dot(h, Wa[...], preferred_element_type=jnp.float32)
                    + ba[...], 0.0)
    h = jnp.dot(h, Wb[...], preferred_element_type=jnp.float32) + bb[...]
    h = jnp.maximum(h, 0.0)
    h = jnp.maximum(jnp.dot(h, Wl1[...], preferred_element_type=jnp.float32)
                    + bl1[...], 0.0)
    h = jnp.maximum(jnp.dot(h, Wl2[...], preferred_element_type=jnp.float32)
                    + bl2[...], 0.0)
    out[...] = jax.nn.sigmoid(h)


def _full(shape):
    return pl.BlockSpec(shape, lambda i: (0, 0))


def _mlp1(p, x, Wa, ba, Wb, bb, B=2000):
    N, D = p.shape[1], p.shape[2]
    H = Wa.shape[1]
    return pl.pallas_call(
        _mlp1_body,
        grid=(N // B,),
        in_specs=[pl.BlockSpec((B, D), lambda i: (i, 0)),
                  pl.BlockSpec((B, D), lambda i: (i, 0)),
                  pl.BlockSpec((B, D), lambda i: (i, 0)),
                  _full((D, H)), _full((1, H)),
                  _full((H, H)), _full((1, H))],
        out_specs=pl.BlockSpec((B, H), lambda i: (i, 0)),
        out_shape=jax.ShapeDtypeStruct((N, H), jnp.float32),
    )(p[0], p[1], x, Wa, ba.reshape(1, -1), Wb, bb.reshape(1, -1))


def _mlp2(p, h1, Wa, ba, Wb, bb, Wl1, bl1, Wl2, bl2, B=2000):
    N, H = p.shape[1], p.shape[2]
    K = Wl1.shape[1]
    return pl.pallas_call(
        _mlp2_body,
        grid=(N // B,),
        in_specs=[pl.BlockSpec((B, H), lambda i: (i, 0)),
                  pl.BlockSpec((B, H), lambda i: (i, 0)),
                  pl.BlockSpec((B, H), lambda i: (i, 0)),
                  _full((H, H)), _full((1, H)),
                  _full((H, H)), _full((1, H)),
                  _full((H, K)), _full((1, K)),
                  _full((K, 1)), _full((1, 1))],
        out_specs=pl.BlockSpec((B, 1), lambda i: (i, 0)),
        out_shape=jax.ShapeDtypeStruct((N, 1), jnp.float32),
    )(p[0], p[1], h1, Wa, ba.reshape(1, -1), Wb, bb.reshape(1, -1),
      Wl1, bl1.reshape(1, -1), Wl2, bl2.reshape(1, -1))


def kernel(x, edge_index, batch, W1a, b1a, W1b, b1b, W2a, b2a, W2b, b2b,
           Wl1, bl1, Wl2, bl2):
    N, D = x.shape
    E = edge_index.shape[1]
    H = W1a.shape[1]
    C = 125
    NW = _NC * _NS
    src3d = edge_index[0].reshape(NW, E // (NW * C), C)
    dst3d = edge_index[1].reshape(NW, E // (NW * C), C)

    scat1 = _make_sc_scatter_add(N, D, E, C, tc_tiling=True)
    scat2 = _make_sc_scatter_add(N, H, E, C, tc_tiling=False)

    p1 = scat1(x, src3d, dst3d)
    h1 = _mlp1(p1, x, W1a, b1a, W1b, b1b)
    p2 = scat2(h1, src3d, dst3d)
    return _mlp2(p2, h1, W2a, b2a, W2b, b2b, Wl1, bl1, Wl2, bl2)
